# Initial kernel scaffold; baseline (speedup 1.0000x reference)
import jax
import jax.numpy as jnp
from jax import lax
from jax.experimental import pallas as pl
from jax.experimental.pallas import tpu as pltpu
from jax.experimental.pallas import tpu_sc as plsc

NC, NS, LANES = 2, 16, 16
NW = NC * NS

B, F_, L_, V_, D_ = 1024, 26, 20, 1000, 64
N = B * F_ * L_
ROWS_PER_W = N // NW
SUB = 128
NSUB = 5
CHUNK = SUB * NSUB
NCHUNK = ROWS_PER_W // CHUNK


def _body(idx_hbm, table_hbm, out_hbm, idx_v, rows_v, sem):
    wid = lax.axis_index("s") * NC + lax.axis_index("c")
    base0 = wid * ROWS_PER_W

    def chunk(g, carry):
        base = base0 + g * CHUNK
        pltpu.sync_copy(idx_hbm.at[pl.ds(base // SUB, NSUB)], idx_v)
        for j in range(NSUB):
            for k in range(SUB // LANES):
                p = base + j * SUB + k * LANES + lax.iota(jnp.int32, (LANES,))
                off = ((p // L_) % F_) * V_
                idx_v[j, pl.ds(k * LANES, LANES)] = (
                    idx_v[j, pl.ds(k * LANES, LANES)] + off
                )
        copies = [
            pltpu.make_async_copy(
                table_hbm.at[idx_v.at[j]],
                rows_v.at[pl.ds(j * SUB, SUB)],
                sem,
            )
            for j in range(NSUB)
        ]
        for c in copies:
            c.start()
        for c in copies:
            c.wait()
        pltpu.sync_copy(rows_v, out_hbm.at[pl.ds(base, CHUNK)])
        return carry

    lax.fori_loop(0, NCHUNK, chunk, 0)


@jax.jit
def kernel(indices, tables):
    flat_tables = tables.reshape(F_ * V_, D_)
    idx2d = indices.reshape(N // SUB, SUB)
    mesh = plsc.VectorSubcoreMesh(
        core_axis_name="c", subcore_axis_name="s", num_cores=NC, num_subcores=NS
    )
    out = pl.kernel(
        _body,
        out_type=jax.ShapeDtypeStruct((N, D_), jnp.float32),
        mesh=mesh,
        scratch_types=[
            pltpu.VMEM((NSUB, SUB), jnp.int32),
            pltpu.VMEM((CHUNK, D_), jnp.float32),
            pltpu.SemaphoreType.DMA,
        ],
    )(idx2d, flat_tables)
    return out.reshape(B, F_ * L_ * D_)


# speedup vs baseline: 7.0880x; 7.0880x over previous
import jax
import jax.numpy as jnp
from jax import lax
from jax.experimental import pallas as pl
from jax.experimental.pallas import tpu as pltpu
from jax.experimental.pallas import tpu_sc as plsc

NC, NS, LANES = 2, 16, 16
NW = NC * NS

B, F_, L_, V_, D_ = 1024, 26, 20, 1000, 64
N = B * F_ * L_
SUB = 128
NSUB = 8
CHUNK = SUB * NSUB
NCHUNK = N // CHUNK
PERIOD = F_ * L_
OFF_LEN = PERIOD + CHUNK


def _body(idx_hbm, table_hbm, off_hbm, out_hbm, idx_v, rows_v, off_v, sem):
    wid = lax.axis_index("s") * NC + lax.axis_index("c")
    pltpu.sync_copy(off_hbm, off_v)
    n_mine = jnp.where(wid < NCHUNK % NW, NCHUNK // NW + 1, NCHUNK // NW)

    def chunk(g, carry):
        base = pl.multiple_of((g * NW + wid) * CHUNK, CHUNK)
        phase = base % PERIOD
        row0 = pl.multiple_of(base // SUB, NSUB)
        pltpu.sync_copy(idx_hbm.at[pl.ds(row0, NSUB)], idx_v)
        for j in range(NSUB):
            for k in range(SUB // LANES):
                off = off_v[pl.ds(phase + j * SUB + k * LANES, LANES)]
                idx_v[j, pl.ds(k * LANES, LANES)] = (
                    idx_v[j, pl.ds(k * LANES, LANES)] + off
                )
        copies = [
            pltpu.make_async_copy(
                table_hbm.at[idx_v.at[j]],
                rows_v.at[pl.ds(j * SUB, SUB)],
                sem,
            )
            for j in range(NSUB)
        ]
        for c in copies:
            c.start()
        for c in copies:
            c.wait()
        pltpu.sync_copy(rows_v, out_hbm.at[pl.ds(base, CHUNK)])
        return carry

    lax.fori_loop(0, n_mine, chunk, 0)


@jax.jit
def kernel(indices, tables):
    flat_tables = tables.reshape(F_ * V_, D_)
    idx2d = indices.reshape(N // SUB, SUB)
    off_lut = (jnp.arange(OFF_LEN, dtype=jnp.int32) // L_ % F_) * V_
    mesh = plsc.VectorSubcoreMesh(
        core_axis_name="c", subcore_axis_name="s", num_cores=NC, num_subcores=NS
    )
    out = pl.kernel(
        _body,
        out_type=jax.ShapeDtypeStruct((N, D_), jnp.float32),
        mesh=mesh,
        scratch_types=[
            pltpu.VMEM((NSUB, SUB), jnp.int32),
            pltpu.VMEM((CHUNK, D_), jnp.float32),
            pltpu.VMEM((OFF_LEN,), jnp.int32),
            pltpu.SemaphoreType.DMA,
        ],
        compiler_params=pltpu.CompilerParams(use_tc_tiling_on_sc=False),
    )(idx2d, flat_tables, off_lut)
    return out.reshape(B, F_ * L_ * D_)


# speedup vs baseline: 7.3301x; 1.0342x over previous
import jax
import jax.numpy as jnp
from jax import lax
from jax.experimental import pallas as pl
from jax.experimental.pallas import tpu as pltpu
from jax.experimental.pallas import tpu_sc as plsc

NC, NS, LANES = 2, 16, 16
NW = NC * NS

B, F_, L_, V_, D_ = 1024, 26, 20, 1000, 64
N = B * F_ * L_
SUB = 128
NSUB = 8
CHUNK = SUB * NSUB
NCHUNK = N // CHUNK
PERIOD = F_ * L_
OFF_LEN = PERIOD + CHUNK


HALF = NSUB // 2


def _body(idx_hbm, table_hbm, off_hbm, out_hbm, idx_v, rows_v, off_v,
          gsem0, gsem1, ssem0, ssem1):
    wid = lax.axis_index("s") * NC + lax.axis_index("c")
    gsems = (gsem0, gsem1)
    ssems = (ssem0, ssem1)
    pltpu.sync_copy(off_hbm, off_v)
    n_mine = jnp.where(wid < NCHUNK % NW, NCHUNK // NW + 1, NCHUNK // NW)

    def make_store(base, h):
        return pltpu.make_async_copy(
            rows_v.at[h],
            out_hbm.at[pl.ds(base + h * (CHUNK // 2), CHUNK // 2)],
            ssems[h],
        )

    def chunk(g, first):
        base = pl.multiple_of((g * NW + wid) * CHUNK, CHUNK)
        phase = base % PERIOD
        row0 = pl.multiple_of(base // SUB, NSUB)
        pltpu.sync_copy(idx_hbm.at[pl.ds(row0, NSUB)], idx_v)
        for j in range(NSUB):
            for k in range(SUB // LANES):
                off = off_v[pl.ds(phase + j * SUB + k * LANES, LANES)]
                idx_v[j, pl.ds(k * LANES, LANES)] = (
                    idx_v[j, pl.ds(k * LANES, LANES)] + off
                )
        for h in (0, 1):
            gathers = [
                pltpu.make_async_copy(
                    table_hbm.at[idx_v.at[h * HALF + j]],
                    rows_v.at[h, pl.ds(j * SUB, SUB)],
                    gsems[h],
                )
                for j in range(HALF)
            ]
            @pl.when(jnp.logical_not(first))
            def _():
                make_store(base, h).wait()
            for c in gathers:
                c.start()
            for c in gathers:
                c.wait()
            make_store(base, h).start()
        return jnp.bool_(False)

    lax.fori_loop(0, n_mine, chunk, jnp.bool_(True))
    for h in (0, 1):
        make_store(0, h).wait()


@jax.jit
def kernel(indices, tables):
    flat_tables = tables.reshape(F_ * V_, D_)
    idx2d = indices.reshape(N // SUB, SUB)
    off_lut = (jnp.arange(OFF_LEN, dtype=jnp.int32) // L_ % F_) * V_
    mesh = plsc.VectorSubcoreMesh(
        core_axis_name="c", subcore_axis_name="s", num_cores=NC, num_subcores=NS
    )
    out = pl.kernel(
        _body,
        out_type=jax.ShapeDtypeStruct((N, D_), jnp.float32),
        mesh=mesh,
        scratch_types=[
            pltpu.VMEM((NSUB, SUB), jnp.int32),
            pltpu.VMEM((2, CHUNK // 2, D_), jnp.float32),
            pltpu.VMEM((OFF_LEN,), jnp.int32),
            pltpu.SemaphoreType.DMA,
            pltpu.SemaphoreType.DMA,
            pltpu.SemaphoreType.DMA,
            pltpu.SemaphoreType.DMA,
        ],
        compiler_params=pltpu.CompilerParams(use_tc_tiling_on_sc=False),
    )(idx2d, flat_tables, off_lut)
    return out.reshape(B, F_ * L_ * D_)
